# baseline (speedup 1.0000x reference)
import functools

import jax
import jax.numpy as jnp
from jax import lax
from jax.experimental import pallas as pl
from jax.experimental.pallas import tpu as pltpu
from jax.experimental.pallas import tpu_sc as plsc

B = 16384
D = 32
G = 4
NC = 2
NS = 16
NW = NC * NS
B_PER_W = B // NW
L = 16


def _sc_gather_body(uidx_hbm, iidx_hbm, ug4, ig4, um4, im4,
                    out_ug, out_ig, out_um, out_im,
                    uidx_v, iidx_v, u4_v, i4_v, g_a, g_b, sem):
    wid = lax.axis_index("s") * NC + lax.axis_index("c")
    base = wid * B_PER_W
    pltpu.sync_copy(uidx_hbm.at[pl.ds(base, B_PER_W)], uidx_v)
    pltpu.sync_copy(iidx_hbm.at[pl.ds(base, B_PER_W)], iidx_v)

    def shift(j):
        s = pl.ds(j * L, L)
        u4_v[s] = lax.shift_right_logical(uidx_v[s], 2)
        i4_v[s] = lax.shift_right_logical(iidx_v[s], 2)

    pl.loop(0, B_PER_W // L)(shift)

    H = B_PER_W // 2
    steps = []
    for tbl, idx, out in ((ug4, u4_v, out_ug), (ig4, i4_v, out_ig),
                          (um4, u4_v, out_um), (im4, i4_v, out_im)):
        for h in range(2):
            steps.append((tbl, idx.at[pl.ds(h * H, H)],
                          out.at[pl.ds(base + h * H, H)]))
    bufs = (g_a, g_b)
    prev = None
    for k, (tbl, idx, out) in enumerate(steps):
        cur = (pltpu.async_copy(tbl.at[idx], bufs[k % 2], sem), out, bufs[k % 2])
        if prev is not None:
            c, pout, pbuf = prev
            c.wait()
            pltpu.sync_copy(pbuf, pout)
        prev = cur
    c, pout, pbuf = prev
    c.wait()
    pltpu.sync_copy(pbuf, pout)


_sc_gather = functools.partial(
    pl.kernel,
    out_type=[jax.ShapeDtypeStruct((B, G * D), jnp.float32)] * 4,
    mesh=plsc.VectorSubcoreMesh(core_axis_name="c", subcore_axis_name="s"),
    compiler_params=pltpu.CompilerParams(use_tc_tiling_on_sc=False),
    scratch_types=[
        pltpu.VMEM((B_PER_W,), jnp.int32),
        pltpu.VMEM((B_PER_W,), jnp.int32),
        pltpu.VMEM((B_PER_W,), jnp.int32),
        pltpu.VMEM((B_PER_W,), jnp.int32),
        pltpu.VMEM((B_PER_W // 2, G * D), jnp.float32),
        pltpu.VMEM((B_PER_W // 2, G * D), jnp.float32),
        pltpu.SemaphoreType.DMA,
    ],
)(_sc_gather_body)


def _dot_t(x, w):
    return lax.dot_general(x, w, (((1,), (1,)), ((), ())),
                           preferred_element_type=jnp.float32)


def _select_group(g, idx):
    k = idx & (G - 1)
    out = jnp.zeros((g.shape[0], D), jnp.float32)
    for j in range(G):
        out = out + jnp.where(k == j, g[:, j * D:(j + 1) * D], 0.0)
    return out


def _tc_dense_body(ug_ref, ig_ref, um_ref, im_ref, u_ref, i_ref,
                   w1a_ref, w1b_ref, b1_ref, w2_ref, b2_ref, w3_ref, b3_ref,
                   wpa_ref, wpb_ref, bp_ref, out_ref):
    u = u_ref[...]
    i = i_ref[...]
    ug = _select_group(ug_ref[...], u)
    ig = _select_group(ig_ref[...], i)
    um = _select_group(um_ref[...], u)
    im = _select_group(im_ref[...], i)
    mf = ug * ig
    h = _dot_t(um, w1a_ref[...]) + _dot_t(im, w1b_ref[...])
    h = jnp.maximum(h + b1_ref[...], 0.0)
    h = jnp.maximum(_dot_t(h, w2_ref[...]) + b2_ref[...], 0.0)
    h = jnp.maximum(_dot_t(h, w3_ref[...]) + b3_ref[...], 0.0)
    out_ref[...] = _dot_t(mf, wpa_ref[...]) + _dot_t(h, wpb_ref[...]) + bp_ref[...]


def kernel(user_indices, item_indices, U_gmf, I_gmf, U_mlp, I_mlp,
           W1, b1, W2, b2, W3, b3, Wp, bp):
    tables4 = [t.reshape(-1, G * D) for t in (U_gmf, I_gmf, U_mlp, I_mlp)]
    ug, ig, um, im = _sc_gather(user_indices, item_indices, *tables4)
    w1a, w1b = W1[:, :D], W1[:, D:]
    wpa, wpb = Wp[:, :D], Wp[:, D:]
    BLK = 4096
    row_spec = lambda w: pl.BlockSpec((BLK, w), lambda r: (r, 0))
    full = lambda a, b: pl.BlockSpec((a, b), lambda r: (0, 0))
    pred = pl.pallas_call(
        _tc_dense_body,
        grid=(B // BLK,),
        in_specs=[row_spec(G * D)] * 4 + [row_spec(1)] * 2 + [
            full(64, D), full(64, D), full(1, 64),
            full(32, 64), full(1, 32),
            full(16, 32), full(1, 16),
            full(1, D), full(1, 16), full(1, 1),
        ],
        out_specs=row_spec(1),
        out_shape=jax.ShapeDtypeStruct((B, 1), jnp.float32),
    )(ug, ig, um, im,
      user_indices.reshape(-1, 1), item_indices.reshape(-1, 1),
      w1a, w1b, b1.reshape(1, -1), W2, b2.reshape(1, -1),
      W3, b3.reshape(1, -1), wpa, wpb, bp.reshape(1, 1))
    return pred.reshape(-1)
